# Initial kernel scaffold; baseline (speedup 1.0000x reference)
import jax
import jax.numpy as jnp
from jax.experimental import pallas as pl


def kernel(feats_final, labels, W1, W2, bn_w, bn_b, lambda_align_K, lambda_align_Z):
    raise NotImplementedError("write your pallas kernel here")


# speedup vs baseline: 1.5761x; 1.5761x over previous
import functools
import jax
import jax.numpy as jnp
from jax import lax
from jax.experimental import pallas as pl

_TOPK = 5
_NEG = -1e9


def _norm_body(x_ref, o_ref):
    x = x_ref[0]
    n = jnp.sqrt(jnp.sum(x * x, axis=1, keepdims=True))
    o_ref[0] = x / jnp.maximum(n, 1e-8)


def _topk_body(xb_ref, xf_ref, labr_ref, labc_ref, ints_ref, vals_ref, *, bm, b_total):
    blk = pl.program_id(1)
    xb = xb_ref[0]
    xf = xf_ref[0]
    sim = lax.dot_general(xb, xf, (((1,), (1,)), ((), ())),
                          preferred_element_type=jnp.float32)
    sim = jnp.clip(sim, -1.0 + 1e-8, 1.0 - 1e-8)
    cols = lax.broadcasted_iota(jnp.int32, (bm, b_total), 1)
    rows_g = lax.broadcasted_iota(jnp.int32, (bm, b_total), 0) + blk * bm
    lab_row = labr_ref[0:1, :]
    lab_blk = labc_ref[:, 0:1]
    cond = (lab_blk == lab_row) & (rows_g != cols) & (sim >= 0.0)
    masked = jnp.where(cond, sim, _NEG)
    cand = jnp.sum((masked > 0.0).astype(jnp.int32), axis=1)
    idx_list, val_list = [], []
    for _ in range(_TOPK):
        mx = jnp.max(masked, axis=1)
        eq = masked == mx[:, None]
        am = jnp.min(jnp.where(eq, cols, b_total), axis=1)
        idx_list.append(am)
        val_list.append(mx)
        masked = jnp.where(cols == am[:, None], -2e9, masked)
    col8 = lax.broadcasted_iota(jnp.int32, (bm, 8), 1)
    ints = jnp.zeros((bm, 8), jnp.int32)
    vals = jnp.zeros((bm, 8), jnp.float32)
    for k in range(_TOPK):
        ints = jnp.where(col8 == k, idx_list[k][:, None], ints)
        vals = jnp.where(col8 == k, val_list[k][:, None], vals)
    ints = jnp.where(col8 == _TOPK, cand[:, None], ints)
    ints_ref[0] = ints
    vals_ref[0] = vals


def _rebuild_rows(ints_blk, vals_blk, intsT, valsT, blk, bm, b_total):
    cols = lax.broadcasted_iota(jnp.int32, (bm, b_total), 1)
    rows_g = lax.broadcasted_iota(jnp.int32, (bm, b_total), 0) + blk * bm
    cand_i = ints_blk[:, _TOPK:_TOPK + 1]
    keep_i = (cand_i >= _TOPK).astype(jnp.float32)
    keep_j = (intsT[_TOPK:_TOPK + 1, :] >= _TOPK).astype(jnp.float32)
    out_m = jnp.zeros((bm, b_total), jnp.bool_)
    a_out = jnp.zeros((bm, b_total), jnp.float32)
    for k in range(_TOPK):
        eq = cols == ints_blk[:, k:k + 1]
        out_m = out_m | eq
        a_out = jnp.where(eq, vals_blk[:, k:k + 1], a_out)
    in_m = jnp.zeros((bm, b_total), jnp.bool_)
    a_in = jnp.zeros((bm, b_total), jnp.float32)
    for k in range(_TOPK):
        eq = intsT[k:k + 1, :] == rows_g
        in_m = in_m | eq
        a_in = jnp.where(eq, valsT[k:k + 1, :] + jnp.zeros((bm, 1), jnp.float32), a_in)
    kf = keep_i * keep_j
    m = jnp.where(out_m | in_m, 1.0, 0.0) * kf
    a = jnp.where(out_m, a_out, a_in) * m
    return m, a


def _deg_body(ints_ref, vals_ref, intsT_ref, valsT_ref, d_ref, *, bm, b_total):
    blk = pl.program_id(1)
    _, a = _rebuild_rows(ints_ref[0], vals_ref[0], intsT_ref[0], valsT_ref[0],
                         blk, bm, b_total)
    d = 1e-6 + jnp.sum(a, axis=1)
    d_ref[0] = jnp.broadcast_to(d[:, None], (bm, 128))


def _loss_body(ints0_ref, vals0_ref, ints1_ref, vals1_ref,
               intsT0_ref, valsT0_ref, intsT1_ref, valsT1_ref,
               d0_ref, d1_ref, dT0_ref, dT1_ref, out_ref, *, bm, b_total):
    blk = pl.program_id(0)
    m0, a0 = _rebuild_rows(ints0_ref[0], vals0_ref[0], intsT0_ref[0],
                           valsT0_ref[0], blk, bm, b_total)
    m1, a1 = _rebuild_rows(ints1_ref[0], vals1_ref[0], intsT1_ref[0],
                           valsT1_ref[0], blk, bm, b_total)
    dinv0_i = lax.rsqrt(d0_ref[0][:, 0:1])
    dinv1_i = lax.rsqrt(d1_ref[0][:, 0:1])
    dinv0_j = lax.rsqrt(dT0_ref[0][0:1, :])
    dinv1_j = lax.rsqrt(dT1_ref[0][0:1, :])
    k0 = dinv0_i * a0 * dinv0_j
    k1 = dinv1_i * a1 * dinv1_j
    meff = jnp.maximum(m0, m1)
    diff = k0 - k1
    num = jnp.sum(diff * diff * meff)
    den = jnp.sum(meff)
    lane = lax.broadcasted_iota(jnp.int32, (1, 128), 1)
    out_ref[0] = jnp.where(lane == 0, num, jnp.where(lane == 1, den, 0.0))


def kernel(feats_final, labels, W1, W2, bn_w, bn_b, lambda_align_K, lambda_align_Z):
    L, B, D = feats_final.shape
    bm1 = 256 if B % 256 == 0 else B
    nb1 = B // bm1
    bm2 = 256 if B % 256 == 0 else B
    nb2 = B // bm2

    xn = pl.pallas_call(
        _norm_body,
        grid=(L,),
        in_specs=[pl.BlockSpec((1, B, D), lambda l: (l, 0, 0))],
        out_specs=pl.BlockSpec((1, B, D), lambda l: (l, 0, 0)),
        out_shape=jax.ShapeDtypeStruct((L, B, D), jnp.float32),
    )(feats_final)

    labr = jnp.broadcast_to(labels[None, :], (8, B))
    labc = jnp.broadcast_to(labels[:, None], (B, 128))

    ints, vals = pl.pallas_call(
        functools.partial(_topk_body, bm=bm1, b_total=B),
        grid=(L, nb1),
        in_specs=[
            pl.BlockSpec((1, bm1, D), lambda l, b: (l, b, 0)),
            pl.BlockSpec((1, B, D), lambda l, b: (l, 0, 0)),
            pl.BlockSpec((8, B), lambda l, b: (0, 0)),
            pl.BlockSpec((bm1, 128), lambda l, b: (b, 0)),
        ],
        out_specs=[
            pl.BlockSpec((1, bm1, 8), lambda l, b: (l, b, 0)),
            pl.BlockSpec((1, bm1, 8), lambda l, b: (l, b, 0)),
        ],
        out_shape=[
            jax.ShapeDtypeStruct((L, B, 8), jnp.int32),
            jax.ShapeDtypeStruct((L, B, 8), jnp.float32),
        ],
    )(xn, xn, labr, labc)

    intsT = jnp.swapaxes(ints, 1, 2)
    valsT = jnp.swapaxes(vals, 1, 2)

    drows = pl.pallas_call(
        functools.partial(_deg_body, bm=bm2, b_total=B),
        grid=(L, nb2),
        in_specs=[
            pl.BlockSpec((1, bm2, 8), lambda l, b: (l, b, 0)),
            pl.BlockSpec((1, bm2, 8), lambda l, b: (l, b, 0)),
            pl.BlockSpec((1, 8, B), lambda l, b: (l, 0, 0)),
            pl.BlockSpec((1, 8, B), lambda l, b: (l, 0, 0)),
        ],
        out_specs=pl.BlockSpec((1, bm2, 128), lambda l, b: (l, b, 0)),
        out_shape=jax.ShapeDtypeStruct((L, B, 128), jnp.float32),
    )(ints, vals, intsT, valsT)

    dT = jnp.broadcast_to(drows[:, None, :, 0], (L, 8, B))

    partial = pl.pallas_call(
        functools.partial(_loss_body, bm=bm2, b_total=B),
        grid=(nb2,),
        in_specs=[
            pl.BlockSpec((1, bm2, 8), lambda b: (0, b, 0)),
            pl.BlockSpec((1, bm2, 8), lambda b: (0, b, 0)),
            pl.BlockSpec((1, bm2, 8), lambda b: (1, b, 0)),
            pl.BlockSpec((1, bm2, 8), lambda b: (1, b, 0)),
            pl.BlockSpec((1, 8, B), lambda b: (0, 0, 0)),
            pl.BlockSpec((1, 8, B), lambda b: (0, 0, 0)),
            pl.BlockSpec((1, 8, B), lambda b: (1, 0, 0)),
            pl.BlockSpec((1, 8, B), lambda b: (1, 0, 0)),
            pl.BlockSpec((1, bm2, 128), lambda b: (0, b, 0)),
            pl.BlockSpec((1, bm2, 128), lambda b: (1, b, 0)),
            pl.BlockSpec((1, 8, B), lambda b: (0, 0, 0)),
            pl.BlockSpec((1, 8, B), lambda b: (1, 0, 0)),
        ],
        out_specs=pl.BlockSpec((1, 1, 128), lambda b: (b, 0, 0)),
        out_shape=jax.ShapeDtypeStruct((nb2, 1, 128), jnp.float32),
    )(ints, vals, ints, vals, intsT, valsT, intsT, valsT, drows, drows, dT, dT)

    num = partial[:, 0, 0].sum()
    den = partial[:, 0, 1].sum()
    loss_align_K = num / jnp.maximum(den, 1e-8)
    loss_align_Z = jnp.zeros((), jnp.float32)
    loss_pga = lambda_align_K * loss_align_K + lambda_align_Z * loss_align_Z
    return (loss_align_K, loss_align_Z, loss_pga)


# speedup vs baseline: 3.5783x; 2.2703x over previous
import functools
import jax
import jax.numpy as jnp
from jax import lax
from jax.experimental import pallas as pl
from jax.experimental.pallas import tpu as pltpu, tpu_sc as plsc

_TOPK = 5
_NEG = -1e9


def _norm_body(x_ref, o_ref):
    x = x_ref[0]
    n = jnp.sqrt(jnp.sum(x * x, axis=1, keepdims=True))
    o_ref[0] = x / jnp.maximum(n, 1e-8)


def _topk_body(xb_ref, xf_ref, labr_ref, labc_ref, ints_ref, vals_ref, *, bm, b_total):
    blk = pl.program_id(1)
    xb = xb_ref[0]
    xf = xf_ref[0]
    sim = lax.dot_general(xb, xf, (((1,), (1,)), ((), ())),
                          preferred_element_type=jnp.float32)
    sim = jnp.clip(sim, -1.0 + 1e-8, 1.0 - 1e-8)
    cols = lax.broadcasted_iota(jnp.int32, (bm, b_total), 1)
    rows_g = lax.broadcasted_iota(jnp.int32, (bm, b_total), 0) + blk * bm
    lab_row = labr_ref[0:1, :]
    lab_blk = labc_ref[:, 0:1]
    cond = (lab_blk == lab_row) & (rows_g != cols) & (sim >= 0.0)
    masked = jnp.where(cond, sim, _NEG)
    cand = jnp.sum((masked > 0.0).astype(jnp.int32), axis=1)
    keepf = jnp.where(cand >= _TOPK, 1.0, 0.0)
    idx_list, val_list = [], []
    for _ in range(_TOPK):
        mx = jnp.max(masked, axis=1)
        eq = masked == mx[:, None]
        am = jnp.min(jnp.where(eq, cols, b_total), axis=1)
        idx_list.append(am)
        val_list.append(mx)
        masked = jnp.where(cols == am[:, None], -2e9, masked)
    col8 = lax.broadcasted_iota(jnp.int32, (bm, 8), 1)
    ints = jnp.zeros((bm, 8), jnp.int32)
    vals = jnp.zeros((bm, 8), jnp.float32)
    for k in range(_TOPK):
        ints = jnp.where(col8 == k, idx_list[k][:, None], ints)
        vals = jnp.where(col8 == k, val_list[k][:, None], vals)
    ints = jnp.where(col8 == _TOPK, cand[:, None], ints)
    vals = jnp.where(col8 == _TOPK, keepf[:, None], vals)
    ints_ref[0] = ints
    vals_ref[0] = vals


def _dinv_body(dp_ref, o_ref):
    x = dp_ref[...]
    d0 = x[0:1, :] + x[1:2, :] + x[2:3, :] + 1e-6
    d1 = x[4:5, :] + x[5:6, :] + x[6:7, :] + 1e-6
    r0 = lax.rsqrt(d0)
    r1 = lax.rsqrt(d1)
    rowi = lax.broadcasted_iota(jnp.int32, x.shape, 0)
    o_ref[...] = jnp.where(rowi == 0, r0, jnp.where(rowi == 1, r1, 0.0))


def _lanes16():
    return lax.broadcasted_iota(jnp.int32, (16,), 0)


def _sc_edges_body(intsT, valsT, intsF, valsF, dpart, wv,
                   iown0, iown1, vown0, vown1, gkeep, gdup, g1i, g1v,
                   obuf, wvbuf, evjb, down0, down1, zbuf, dsh0, dsh1,
                   *, b_total, rp, nc, ns):
    cid = lax.axis_index("c")
    sid = lax.axis_index("s")
    wid = sid * nc + cid
    base = wid * rp
    nch = rp // 16
    lanes = _lanes16()
    z16 = jnp.zeros((16,), jnp.float32)

    def zb(i, c):
        zbuf[pl.ds(i * 16, 16)] = z16
        return c
    lax.fori_loop(0, b_total // 16, zb, 0)

    @pl.when(sid == 0)
    def _():
        pltpu.sync_copy(zbuf, dsh0)
        pltpu.sync_copy(zbuf, dsh1)
    plsc.subcore_barrier()

    pltpu.sync_copy(intsT.at[0, :, pl.ds(base, rp)], iown0)
    pltpu.sync_copy(intsT.at[1, :, pl.ds(base, rp)], iown1)
    pltpu.sync_copy(valsT.at[0, :, pl.ds(base, rp)], vown0)
    pltpu.sync_copy(valsT.at[1, :, pl.ds(base, rp)], vown1)

    def zd(i, c):
        down0[pl.ds(i * 16, 16)] = z16
        down1[pl.ds(i * 16, 16)] = z16
        return c
    lax.fori_loop(0, nch, zd, 0)

    for l in range(2):
        iown = iown0 if l == 0 else iown1
        vown = vown0 if l == 0 else vown1
        dsh = dsh0 if l == 0 else dsh1
        down = down0 if l == 0 else down1
        for k in range(_TOPK):
            def ob(c, carry):
                j16 = iown[k, pl.ds(c * 16, 16)]
                for kk in range(_TOPK):
                    obuf[kk, pl.ds(c * 16, 16)] = j16 + (l * 8 + kk) * b_total
                obuf[5, pl.ds(c * 16, 16)] = j16 + (l * 8 + 5) * b_total
                if l == 0:
                    for kk in range(_TOPK):
                        obuf[6 + kk, pl.ds(c * 16, 16)] = j16 + (8 + kk) * b_total
                    obuf[11, pl.ds(c * 16, 16)] = j16 + (8 + 5) * b_total
                return carry
            lax.fori_loop(0, nch, ob, 0)

            for kk in range(_TOPK):
                pltpu.sync_copy(intsF.at[obuf.at[kk]], gdup.at[kk])
            pltpu.sync_copy(valsF.at[obuf.at[5]], gkeep)
            if l == 0:
                for kk in range(_TOPK):
                    pltpu.sync_copy(intsF.at[obuf.at[6 + kk]], g1i.at[kk])
                    pltpu.sync_copy(valsF.at[obuf.at[6 + kk]], g1v.at[kk])
                pltpu.sync_copy(valsF.at[obuf.at[11]], g1v.at[5])

            def ch(c, carry):
                sl = pl.ds(c * 16, 16)
                i16 = base + c * 16 + lanes
                j16 = iown[k, sl]
                v16 = vown[k, sl]
                valid = vown[5, sl] * gkeep[sl]
                dup_b = (gdup[0, sl] == i16)
                for kk in range(1, _TOPK):
                    dup_b = dup_b | (gdup[kk, sl] == i16)
                dupf = jnp.where(dup_b, 1.0, 0.0)
                wvbuf[k, sl] = (2.0 - dupf) * valid
                ev = v16 * valid
                down[sl] = down[sl] + ev
                evjb[sl] = ev * (1.0 - dupf)
                if l == 0:
                    fi_b = (iown1[0, sl] == j16)
                    vfi = jnp.where(iown1[0, sl] == j16, vown1[0, sl], 0.0)
                    for kk in range(1, _TOPK):
                        m = iown1[kk, sl] == j16
                        fi_b = fi_b | m
                        vfi = jnp.where(m, vown1[kk, sl], vfi)
                    fj_b = (g1i[0, sl] == i16)
                    vfj = jnp.where(g1i[0, sl] == i16, g1v[0, sl], 0.0)
                    for kk in range(1, _TOPK):
                        m = g1i[kk, sl] == i16
                        fj_b = fj_b | m
                        vfj = jnp.where(m, g1v[kk, sl], vfj)
                    in1 = jnp.where(fi_b | fj_b, 1.0, 0.0) * vown1[5, sl] * g1v[5, sl]
                    wvbuf[5 + k, sl] = in1
                    wvbuf[10 + k, sl] = jnp.where(fi_b, vfi, vfj)
                return carry
            lax.fori_loop(0, nch, ch, 0)

            pltpu.sync_copy(evjb, dsh.at[iown.at[k]], add=True)

        pltpu.sync_copy(wvbuf, wv.at[l, :, pl.ds(base, rp)])
        pltpu.sync_copy(down, dpart.at[l, 2, pl.ds(base, rp)])

    plsc.subcore_barrier()
    rp2 = b_total // ns
    sbase = sid * rp2
    pltpu.sync_copy(dsh0.at[pl.ds(sbase, rp2)], dpart.at[0, cid, pl.ds(sbase, rp2)])
    pltpu.sync_copy(dsh1.at[pl.ds(sbase, rp2)], dpart.at[1, cid, pl.ds(sbase, rp2)])


def _sc_loss_body(intsT, valsT, wv, dinv8, dinv8F, out,
                  iown0, iown1, vown0, vown1, wvb0, wvb1,
                  dvi0, dvi1, gdj, gdj1, obuf, accb,
                  *, b_total, rp, nc):
    cid = lax.axis_index("c")
    sid = lax.axis_index("s")
    wid = sid * nc + cid
    base = wid * rp
    nch = rp // 16
    z16 = jnp.zeros((16,), jnp.float32)

    pltpu.sync_copy(intsT.at[0, :, pl.ds(base, rp)], iown0)
    pltpu.sync_copy(intsT.at[1, :, pl.ds(base, rp)], iown1)
    pltpu.sync_copy(valsT.at[0, :, pl.ds(base, rp)], vown0)
    pltpu.sync_copy(valsT.at[1, :, pl.ds(base, rp)], vown1)
    pltpu.sync_copy(wv.at[0, :, pl.ds(base, rp)], wvb0)
    pltpu.sync_copy(wv.at[1, :, pl.ds(base, rp)], wvb1)
    pltpu.sync_copy(dinv8.at[0, pl.ds(base, rp)], dvi0)
    pltpu.sync_copy(dinv8.at[1, pl.ds(base, rp)], dvi1)

    s0 = z16; n0 = z16; s1 = z16; n1 = z16; cx = z16; ncx = z16
    for l in range(2):
        iown = iown0 if l == 0 else iown1
        vown = vown0 if l == 0 else vown1
        wvb = wvb0 if l == 0 else wvb1
        dvi = dvi0 if l == 0 else dvi1
        for k in range(_TOPK):
            def ob(c, carry):
                j16 = iown[k, pl.ds(c * 16, 16)]
                obuf[0, pl.ds(c * 16, 16)] = j16 + l * b_total
                if l == 0:
                    obuf[1, pl.ds(c * 16, 16)] = j16 + b_total
                return carry
            lax.fori_loop(0, nch, ob, 0)
            pltpu.sync_copy(dinv8F.at[obuf.at[0]], gdj)
            if l == 0:
                pltpu.sync_copy(dinv8F.at[obuf.at[1]], gdj1)

            def ch(c, carry):
                a_s, a_n, a_c, a_nc = carry
                sl = pl.ds(c * 16, 16)
                wvk = wvb[k, sl]
                ke = dvi[sl] * vown[k, sl] * gdj[sl]
                a_s = a_s + wvk * ke * ke
                a_n = a_n + wvk
                if l == 0:
                    in1 = wvb[5 + k, sl]
                    v1e = wvb[10 + k, sl]
                    k1x = dvi1[sl] * v1e * gdj1[sl]
                    t = wvk * in1
                    a_c = a_c + t * ke * k1x
                    a_nc = a_nc + t
                return (a_s, a_n, a_c, a_nc)

            if l == 0:
                s0, n0, cx, ncx = lax.fori_loop(0, nch, ch, (s0, n0, cx, ncx))
            else:
                s1, n1, cx, ncx = lax.fori_loop(0, nch, ch, (s1, n1, cx, ncx))

    accb[pl.ds(0, 16)] = s0
    accb[pl.ds(16, 16)] = n0
    accb[pl.ds(32, 16)] = s1
    accb[pl.ds(48, 16)] = n1
    accb[pl.ds(64, 16)] = cx
    accb[pl.ds(80, 16)] = ncx
    accb[pl.ds(96, 16)] = z16
    accb[pl.ds(112, 16)] = z16
    pltpu.sync_copy(accb, out.at[wid])


def kernel(feats_final, labels, W1, W2, bn_w, bn_b, lambda_align_K, lambda_align_Z):
    L, B, D = feats_final.shape
    bm1 = 256 if B % 256 == 0 else B
    nb1 = B // bm1

    info = plsc.get_sparse_core_info()
    nc, ns = info.num_cores, info.num_subcores
    nw = nc * ns
    rp = B // nw

    xn = pl.pallas_call(
        _norm_body,
        grid=(L,),
        in_specs=[pl.BlockSpec((1, B, D), lambda l: (l, 0, 0))],
        out_specs=pl.BlockSpec((1, B, D), lambda l: (l, 0, 0)),
        out_shape=jax.ShapeDtypeStruct((L, B, D), jnp.float32),
    )(feats_final)

    labr = jnp.broadcast_to(labels[None, :], (8, B))
    labc = jnp.broadcast_to(labels[:, None], (B, 128))

    ints, vals = pl.pallas_call(
        functools.partial(_topk_body, bm=bm1, b_total=B),
        grid=(L, nb1),
        in_specs=[
            pl.BlockSpec((1, bm1, D), lambda l, b: (l, b, 0)),
            pl.BlockSpec((1, B, D), lambda l, b: (l, 0, 0)),
            pl.BlockSpec((8, B), lambda l, b: (0, 0)),
            pl.BlockSpec((bm1, 128), lambda l, b: (b, 0)),
        ],
        out_specs=[
            pl.BlockSpec((1, bm1, 8), lambda l, b: (l, b, 0)),
            pl.BlockSpec((1, bm1, 8), lambda l, b: (l, b, 0)),
        ],
        out_shape=[
            jax.ShapeDtypeStruct((L, B, 8), jnp.int32),
            jax.ShapeDtypeStruct((L, B, 8), jnp.float32),
        ],
    )(xn, xn, labr, labc)

    intsT = jnp.swapaxes(ints, 1, 2)
    valsT = jnp.swapaxes(vals, 1, 2)
    intsF = intsT.reshape(-1)
    valsF = valsT.reshape(-1)

    mesh = plsc.VectorSubcoreMesh(core_axis_name="c", subcore_axis_name="s")
    dpart, wv = pl.kernel(
        functools.partial(_sc_edges_body, b_total=B, rp=rp, nc=nc, ns=ns),
        out_type=[
            jax.ShapeDtypeStruct((L, 4, B), jnp.float32),
            jax.ShapeDtypeStruct((L, 16, B), jnp.float32),
        ],
        mesh=mesh,
        scratch_types=[
            pltpu.VMEM((8, rp), jnp.int32),
            pltpu.VMEM((8, rp), jnp.int32),
            pltpu.VMEM((8, rp), jnp.float32),
            pltpu.VMEM((8, rp), jnp.float32),
            pltpu.VMEM((rp,), jnp.float32),
            pltpu.VMEM((8, rp), jnp.int32),
            pltpu.VMEM((8, rp), jnp.int32),
            pltpu.VMEM((8, rp), jnp.float32),
            pltpu.VMEM((12, rp), jnp.int32),
            pltpu.VMEM((16, rp), jnp.float32),
            pltpu.VMEM((rp,), jnp.float32),
            pltpu.VMEM((rp,), jnp.float32),
            pltpu.VMEM((rp,), jnp.float32),
            pltpu.VMEM((B,), jnp.float32),
            pltpu.MemorySpace.VMEM_SHARED((B,), jnp.float32),
            pltpu.MemorySpace.VMEM_SHARED((B,), jnp.float32),
        ],
    )(intsT, valsT, intsF, valsF)

    dinv8 = pl.pallas_call(
        _dinv_body,
        grid=(1,),
        in_specs=[pl.BlockSpec((8, B), lambda i: (0, 0))],
        out_specs=pl.BlockSpec((8, B), lambda i: (0, 0)),
        out_shape=jax.ShapeDtypeStruct((8, B), jnp.float32),
    )(dpart.reshape(8, B))

    partial = pl.kernel(
        functools.partial(_sc_loss_body, b_total=B, rp=rp, nc=nc),
        out_type=jax.ShapeDtypeStruct((nw, 128), jnp.float32),
        mesh=mesh,
        scratch_types=[
            pltpu.VMEM((8, rp), jnp.int32),
            pltpu.VMEM((8, rp), jnp.int32),
            pltpu.VMEM((8, rp), jnp.float32),
            pltpu.VMEM((8, rp), jnp.float32),
            pltpu.VMEM((16, rp), jnp.float32),
            pltpu.VMEM((16, rp), jnp.float32),
            pltpu.VMEM((rp,), jnp.float32),
            pltpu.VMEM((rp,), jnp.float32),
            pltpu.VMEM((rp,), jnp.float32),
            pltpu.VMEM((rp,), jnp.float32),
            pltpu.VMEM((2, rp), jnp.int32),
            pltpu.VMEM((128,), jnp.float32),
        ],
    )(intsT, valsT, wv, dinv8, dinv8.reshape(-1))

    psum = partial.reshape(nw, 8, 16)[:, :6, :].sum(axis=(0, 2))
    num = psum[0] + psum[2] - 2.0 * psum[4]
    den = jnp.maximum(psum[1] + psum[3] - psum[5], 1e-8)
    loss_align_K = num / den
    loss_align_Z = jnp.zeros((), jnp.float32)
    loss_pga = lambda_align_K * loss_align_K + lambda_align_Z * loss_align_Z
    return (loss_align_K, loss_align_Z, loss_pga)


# speedup vs baseline: 4.1779x; 1.1676x over previous
import functools
import jax
import jax.numpy as jnp
from jax import lax
from jax.experimental import pallas as pl
from jax.experimental.pallas import tpu as pltpu, tpu_sc as plsc

_TOPK = 5
_NEG = -1e9


def _norm_body(x_ref, o_ref):
    x = x_ref[0]
    n = jnp.sqrt(jnp.sum(x * x, axis=1, keepdims=True))
    o_ref[0] = x / jnp.maximum(n, 1e-8)


def _topk_body(xb_ref, xf_ref, labr_ref, labc_ref, ints_ref, vals_ref, *, bm, b_total):
    blk = pl.program_id(1)
    xb = xb_ref[0]
    xf = xf_ref[0]
    sim = lax.dot_general(xb, xf, (((1,), (1,)), ((), ())),
                          preferred_element_type=jnp.float32)
    sim = jnp.clip(sim, -1.0 + 1e-8, 1.0 - 1e-8)
    cols = lax.broadcasted_iota(jnp.int32, (bm, b_total), 1)
    rows_g = lax.broadcasted_iota(jnp.int32, (bm, b_total), 0) + blk * bm
    lab_row = labr_ref[0:1, :]
    lab_blk = labc_ref[:, 0:1]
    cond = (lab_blk == lab_row) & (rows_g != cols) & (sim >= 0.0)
    masked = jnp.where(cond, sim, _NEG)
    cand = jnp.sum((masked > 0.0).astype(jnp.int32), axis=1)
    keepf = jnp.where(cand >= _TOPK, 1.0, 0.0)
    idx_list, val_list = [], []
    for _ in range(_TOPK):
        mx = jnp.max(masked, axis=1)
        eq = masked == mx[:, None]
        am = jnp.min(jnp.where(eq, cols, b_total), axis=1)
        idx_list.append(am)
        val_list.append(mx)
        masked = jnp.where(cols == am[:, None], -2e9, masked)
    col8 = lax.broadcasted_iota(jnp.int32, (bm, 8), 1)
    ints = jnp.zeros((bm, 8), jnp.int32)
    vals = jnp.zeros((bm, 8), jnp.float32)
    for k in range(_TOPK):
        ints = jnp.where(col8 == k, idx_list[k][:, None], ints)
        vals = jnp.where(col8 == k, val_list[k][:, None], vals)
    ints = jnp.where(col8 == _TOPK, cand[:, None], ints)
    vals = jnp.where(col8 == _TOPK, keepf[:, None], vals)
    ints_ref[0] = ints
    vals_ref[0] = vals


def _dinv_body(dp_ref, o_ref):
    x = dp_ref[...]
    d0 = x[0:1, :] + x[1:2, :] + x[2:3, :] + 1e-6
    d1 = x[4:5, :] + x[5:6, :] + x[6:7, :] + 1e-6
    r0 = lax.rsqrt(d0)
    r1 = lax.rsqrt(d1)
    rowi = lax.broadcasted_iota(jnp.int32, x.shape, 0)
    o_ref[...] = jnp.where(rowi == 0, r0, jnp.where(rowi == 1, r1, 0.0))


def _lanes16():
    return lax.broadcasted_iota(jnp.int32, (16,), 0)


def _sc_edges_body(intsT, valsT, intsF, valsF, dpart, wv,
                   iown0, iown1, vown0, vown1, gkeep, gdup, g1i, g1v,
                   obuf, wvbuf, evjb, down0, down1, zbuf, dsh0, dsh1, gsem,
                   *, b_total, rp, nc, ns):
    cid = lax.axis_index("c")
    sid = lax.axis_index("s")
    wid = sid * nc + cid
    base = wid * rp
    nch = rp // 16
    lanes = _lanes16()
    z16 = jnp.zeros((16,), jnp.float32)

    def zb(i, c):
        zbuf[pl.ds(i * 16, 16)] = z16
        return c
    lax.fori_loop(0, b_total // 16, zb, 0)

    @pl.when(sid == 0)
    def _():
        pltpu.sync_copy(zbuf, dsh0)
        pltpu.sync_copy(zbuf, dsh1)
    plsc.subcore_barrier()

    pltpu.sync_copy(intsT.at[0, :, pl.ds(base, rp)], iown0)
    pltpu.sync_copy(intsT.at[1, :, pl.ds(base, rp)], iown1)
    pltpu.sync_copy(valsT.at[0, :, pl.ds(base, rp)], vown0)
    pltpu.sync_copy(valsT.at[1, :, pl.ds(base, rp)], vown1)

    def zd(i, c):
        down0[pl.ds(i * 16, 16)] = z16
        down1[pl.ds(i * 16, 16)] = z16
        return c
    lax.fori_loop(0, nch, zd, 0)

    for l in range(2):
        iown = iown0 if l == 0 else iown1
        vown = vown0 if l == 0 else vown1
        dsh = dsh0 if l == 0 else dsh1
        down = down0 if l == 0 else down1
        for k in range(_TOPK):
            def ob(c, carry):
                j16 = iown[k, pl.ds(c * 16, 16)]
                for kk in range(_TOPK):
                    obuf[kk, pl.ds(c * 16, 16)] = j16 + (l * 8 + kk) * b_total
                obuf[5, pl.ds(c * 16, 16)] = j16 + (l * 8 + 5) * b_total
                if l == 0:
                    for kk in range(_TOPK):
                        obuf[6 + kk, pl.ds(c * 16, 16)] = j16 + (8 + kk) * b_total
                    obuf[11, pl.ds(c * 16, 16)] = j16 + (8 + 5) * b_total
                return carry
            lax.fori_loop(0, nch, ob, 0)

            handles = []
            for kk in range(_TOPK):
                handles.append(pltpu.async_copy(intsF.at[obuf.at[kk]], gdup.at[kk], gsem))
            handles.append(pltpu.async_copy(valsF.at[obuf.at[5]], gkeep, gsem))
            if l == 0:
                for kk in range(_TOPK):
                    handles.append(pltpu.async_copy(intsF.at[obuf.at[6 + kk]], g1i.at[kk], gsem))
                    handles.append(pltpu.async_copy(valsF.at[obuf.at[6 + kk]], g1v.at[kk], gsem))
                handles.append(pltpu.async_copy(valsF.at[obuf.at[11]], g1v.at[5], gsem))
            for h in handles:
                h.wait()

            def ch(c, carry):
                sl = pl.ds(c * 16, 16)
                i16 = base + c * 16 + lanes
                j16 = iown[k, sl]
                v16 = vown[k, sl]
                valid = vown[5, sl] * gkeep[sl]
                dup_b = (gdup[0, sl] == i16)
                for kk in range(1, _TOPK):
                    dup_b = dup_b | (gdup[kk, sl] == i16)
                dupf = jnp.where(dup_b, 1.0, 0.0)
                wvbuf[k, sl] = (2.0 - dupf) * valid
                ev = v16 * valid
                down[sl] = down[sl] + ev
                evjb[sl] = ev * (1.0 - dupf)
                if l == 0:
                    fi_b = (iown1[0, sl] == j16)
                    vfi = jnp.where(iown1[0, sl] == j16, vown1[0, sl], 0.0)
                    for kk in range(1, _TOPK):
                        m = iown1[kk, sl] == j16
                        fi_b = fi_b | m
                        vfi = jnp.where(m, vown1[kk, sl], vfi)
                    fj_b = (g1i[0, sl] == i16)
                    vfj = jnp.where(g1i[0, sl] == i16, g1v[0, sl], 0.0)
                    for kk in range(1, _TOPK):
                        m = g1i[kk, sl] == i16
                        fj_b = fj_b | m
                        vfj = jnp.where(m, g1v[kk, sl], vfj)
                    in1 = jnp.where(fi_b | fj_b, 1.0, 0.0) * vown1[5, sl] * g1v[5, sl]
                    wvbuf[5 + k, sl] = in1
                    wvbuf[10 + k, sl] = jnp.where(fi_b, vfi, vfj)
                return carry
            lax.fori_loop(0, nch, ch, 0)

            pltpu.sync_copy(evjb, dsh.at[iown.at[k]], add=True)

        pltpu.sync_copy(wvbuf, wv.at[l, :, pl.ds(base, rp)])
        pltpu.sync_copy(down, dpart.at[l, 2, pl.ds(base, rp)])

    plsc.subcore_barrier()
    rp2 = b_total // ns
    sbase = sid * rp2
    pltpu.sync_copy(dsh0.at[pl.ds(sbase, rp2)], dpart.at[0, cid, pl.ds(sbase, rp2)])
    pltpu.sync_copy(dsh1.at[pl.ds(sbase, rp2)], dpart.at[1, cid, pl.ds(sbase, rp2)])


def _sc_loss_body(intsT, valsT, wv, dinv8, dinv8F, out,
                  iown0, iown1, vown0, vown1, wvb0, wvb1,
                  dvi0, dvi1, gdj, gdj1, obuf, accb, gsem,
                  *, b_total, rp, nc):
    cid = lax.axis_index("c")
    sid = lax.axis_index("s")
    wid = sid * nc + cid
    base = wid * rp
    nch = rp // 16
    z16 = jnp.zeros((16,), jnp.float32)

    pltpu.sync_copy(intsT.at[0, :, pl.ds(base, rp)], iown0)
    pltpu.sync_copy(intsT.at[1, :, pl.ds(base, rp)], iown1)
    pltpu.sync_copy(valsT.at[0, :, pl.ds(base, rp)], vown0)
    pltpu.sync_copy(valsT.at[1, :, pl.ds(base, rp)], vown1)
    pltpu.sync_copy(wv.at[0, :, pl.ds(base, rp)], wvb0)
    pltpu.sync_copy(wv.at[1, :, pl.ds(base, rp)], wvb1)
    pltpu.sync_copy(dinv8.at[0, pl.ds(base, rp)], dvi0)
    pltpu.sync_copy(dinv8.at[1, pl.ds(base, rp)], dvi1)

    s0 = z16; n0 = z16; s1 = z16; n1 = z16; cx = z16; ncx = z16
    for l in range(2):
        iown = iown0 if l == 0 else iown1
        vown = vown0 if l == 0 else vown1
        wvb = wvb0 if l == 0 else wvb1
        dvi = dvi0 if l == 0 else dvi1
        for k in range(_TOPK):
            def ob(c, carry):
                j16 = iown[k, pl.ds(c * 16, 16)]
                obuf[0, pl.ds(c * 16, 16)] = j16 + l * b_total
                if l == 0:
                    obuf[1, pl.ds(c * 16, 16)] = j16 + b_total
                return carry
            lax.fori_loop(0, nch, ob, 0)
            hs = [pltpu.async_copy(dinv8F.at[obuf.at[0]], gdj, gsem)]
            if l == 0:
                hs.append(pltpu.async_copy(dinv8F.at[obuf.at[1]], gdj1, gsem))
            for h in hs:
                h.wait()

            def ch(c, carry):
                a_s, a_n, a_c, a_nc = carry
                sl = pl.ds(c * 16, 16)
                wvk = wvb[k, sl]
                ke = dvi[sl] * vown[k, sl] * gdj[sl]
                a_s = a_s + wvk * ke * ke
                a_n = a_n + wvk
                if l == 0:
                    in1 = wvb[5 + k, sl]
                    v1e = wvb[10 + k, sl]
                    k1x = dvi1[sl] * v1e * gdj1[sl]
                    t = wvk * in1
                    a_c = a_c + t * ke * k1x
                    a_nc = a_nc + t
                return (a_s, a_n, a_c, a_nc)

            if l == 0:
                s0, n0, cx, ncx = lax.fori_loop(0, nch, ch, (s0, n0, cx, ncx))
            else:
                s1, n1, cx, ncx = lax.fori_loop(0, nch, ch, (s1, n1, cx, ncx))

    accb[pl.ds(0, 16)] = s0
    accb[pl.ds(16, 16)] = n0
    accb[pl.ds(32, 16)] = s1
    accb[pl.ds(48, 16)] = n1
    accb[pl.ds(64, 16)] = cx
    accb[pl.ds(80, 16)] = ncx
    accb[pl.ds(96, 16)] = z16
    accb[pl.ds(112, 16)] = z16
    pltpu.sync_copy(accb, out.at[wid])


def kernel(feats_final, labels, W1, W2, bn_w, bn_b, lambda_align_K, lambda_align_Z):
    L, B, D = feats_final.shape
    bm1 = 256 if B % 256 == 0 else B
    nb1 = B // bm1

    info = plsc.get_sparse_core_info()
    nc, ns = info.num_cores, info.num_subcores
    nw = nc * ns
    rp = B // nw

    xn = pl.pallas_call(
        _norm_body,
        grid=(L,),
        in_specs=[pl.BlockSpec((1, B, D), lambda l: (l, 0, 0))],
        out_specs=pl.BlockSpec((1, B, D), lambda l: (l, 0, 0)),
        out_shape=jax.ShapeDtypeStruct((L, B, D), jnp.float32),
    )(feats_final)

    labr = jnp.broadcast_to(labels[None, :], (8, B))
    labc = jnp.broadcast_to(labels[:, None], (B, 128))

    ints, vals = pl.pallas_call(
        functools.partial(_topk_body, bm=bm1, b_total=B),
        grid=(L, nb1),
        in_specs=[
            pl.BlockSpec((1, bm1, D), lambda l, b: (l, b, 0)),
            pl.BlockSpec((1, B, D), lambda l, b: (l, 0, 0)),
            pl.BlockSpec((8, B), lambda l, b: (0, 0)),
            pl.BlockSpec((bm1, 128), lambda l, b: (b, 0)),
        ],
        out_specs=[
            pl.BlockSpec((1, bm1, 8), lambda l, b: (l, b, 0)),
            pl.BlockSpec((1, bm1, 8), lambda l, b: (l, b, 0)),
        ],
        out_shape=[
            jax.ShapeDtypeStruct((L, B, 8), jnp.int32),
            jax.ShapeDtypeStruct((L, B, 8), jnp.float32),
        ],
    )(xn, xn, labr, labc)

    intsT = jnp.swapaxes(ints, 1, 2)
    valsT = jnp.swapaxes(vals, 1, 2)
    intsF = intsT.reshape(-1)
    valsF = valsT.reshape(-1)

    mesh = plsc.VectorSubcoreMesh(core_axis_name="c", subcore_axis_name="s")
    dpart, wv = pl.kernel(
        functools.partial(_sc_edges_body, b_total=B, rp=rp, nc=nc, ns=ns),
        out_type=[
            jax.ShapeDtypeStruct((L, 4, B), jnp.float32),
            jax.ShapeDtypeStruct((L, 16, B), jnp.float32),
        ],
        mesh=mesh,
        scratch_types=[
            pltpu.VMEM((8, rp), jnp.int32),
            pltpu.VMEM((8, rp), jnp.int32),
            pltpu.VMEM((8, rp), jnp.float32),
            pltpu.VMEM((8, rp), jnp.float32),
            pltpu.VMEM((rp,), jnp.float32),
            pltpu.VMEM((8, rp), jnp.int32),
            pltpu.VMEM((8, rp), jnp.int32),
            pltpu.VMEM((8, rp), jnp.float32),
            pltpu.VMEM((12, rp), jnp.int32),
            pltpu.VMEM((16, rp), jnp.float32),
            pltpu.VMEM((rp,), jnp.float32),
            pltpu.VMEM((rp,), jnp.float32),
            pltpu.VMEM((rp,), jnp.float32),
            pltpu.VMEM((B,), jnp.float32),
            pltpu.MemorySpace.VMEM_SHARED((B,), jnp.float32),
            pltpu.MemorySpace.VMEM_SHARED((B,), jnp.float32),
            pltpu.SemaphoreType.DMA,
        ],
    )(intsT, valsT, intsF, valsF)

    dinv8 = pl.pallas_call(
        _dinv_body,
        grid=(1,),
        in_specs=[pl.BlockSpec((8, B), lambda i: (0, 0))],
        out_specs=pl.BlockSpec((8, B), lambda i: (0, 0)),
        out_shape=jax.ShapeDtypeStruct((8, B), jnp.float32),
    )(dpart.reshape(8, B))

    partial = pl.kernel(
        functools.partial(_sc_loss_body, b_total=B, rp=rp, nc=nc),
        out_type=jax.ShapeDtypeStruct((nw, 128), jnp.float32),
        mesh=mesh,
        scratch_types=[
            pltpu.VMEM((8, rp), jnp.int32),
            pltpu.VMEM((8, rp), jnp.int32),
            pltpu.VMEM((8, rp), jnp.float32),
            pltpu.VMEM((8, rp), jnp.float32),
            pltpu.VMEM((16, rp), jnp.float32),
            pltpu.VMEM((16, rp), jnp.float32),
            pltpu.VMEM((rp,), jnp.float32),
            pltpu.VMEM((rp,), jnp.float32),
            pltpu.VMEM((rp,), jnp.float32),
            pltpu.VMEM((rp,), jnp.float32),
            pltpu.VMEM((2, rp), jnp.int32),
            pltpu.VMEM((128,), jnp.float32),
            pltpu.SemaphoreType.DMA,
        ],
    )(intsT, valsT, wv, dinv8, dinv8.reshape(-1))

    psum = partial.reshape(nw, 8, 16)[:, :6, :].sum(axis=(0, 2))
    num = psum[0] + psum[2] - 2.0 * psum[4]
    den = jnp.maximum(psum[1] + psum[3] - psum[5], 1e-8)
    loss_align_K = num / den
    loss_align_Z = jnp.zeros((), jnp.float32)
    loss_pga = lambda_align_K * loss_align_K + lambda_align_Z * loss_align_Z
    return (loss_align_K, loss_align_Z, loss_pga)


# speedup vs baseline: 5.6449x; 1.3512x over previous
import functools
import jax
import jax.numpy as jnp
from jax import lax
from jax.experimental import pallas as pl
from jax.experimental.pallas import tpu as pltpu, tpu_sc as plsc

_TOPK = 5
_NEG = -1e9


def _norm_body(x_ref, o_ref):
    x = x_ref[0]
    n = jnp.sqrt(jnp.sum(x * x, axis=1, keepdims=True))
    o_ref[0] = x / jnp.maximum(n, 1e-8)


def _topk_body(xb_ref, xf_ref, labr_ref, labc_ref, ints_ref, vals_ref, *, bm, b_total):
    blk = pl.program_id(1)
    xb = xb_ref[0]
    xf = xf_ref[0]
    sim = lax.dot_general(xb, xf, (((1,), (1,)), ((), ())),
                          preferred_element_type=jnp.float32)
    sim = jnp.clip(sim, -1.0 + 1e-8, 1.0 - 1e-8)
    cols = lax.broadcasted_iota(jnp.int32, (bm, b_total), 1)
    rows_g = lax.broadcasted_iota(jnp.int32, (bm, b_total), 0) + blk * bm
    lab_row = labr_ref[0:1, :]
    lab_blk = labc_ref[:, 0:1]
    cond = (lab_blk == lab_row) & (rows_g != cols) & (sim >= 0.0)
    cand = jnp.sum((cond & (sim > 0.0)).astype(jnp.int32), axis=1)
    keepf = jnp.where(cand >= _TOPK, 1.0, 0.0)
    bits = lax.bitcast_convert_type(sim, jnp.int32)
    pb = (bits & jnp.int32(-4096)) | (b_total - 1 - cols)
    packed = jnp.where(cond, lax.bitcast_convert_type(pb, jnp.float32), _NEG)
    idx_list, val_list = [], []
    for _ in range(_TOPK):
        pmax = jnp.max(packed, axis=1)
        pbits = lax.bitcast_convert_type(pmax, jnp.int32)
        am = (b_total - 1) - (pbits & jnp.int32(4095))
        mx = lax.bitcast_convert_type(pbits & jnp.int32(-4096), jnp.float32)
        idx_list.append(am)
        val_list.append(mx)
        packed = jnp.where(packed == pmax[:, None], -2e9, packed)
    col8 = lax.broadcasted_iota(jnp.int32, (bm, 8), 1)
    ints = jnp.zeros((bm, 8), jnp.int32)
    vals = jnp.zeros((bm, 8), jnp.float32)
    for k in range(_TOPK):
        ints = jnp.where(col8 == k, idx_list[k][:, None], ints)
        vals = jnp.where(col8 == k, val_list[k][:, None], vals)
    ints = jnp.where(col8 == _TOPK, cand[:, None], ints)
    vals = jnp.where(col8 == _TOPK, keepf[:, None], vals)
    ints_ref[0] = ints
    vals_ref[0] = vals


def _dinv_body(dp_ref, o_ref):
    x = dp_ref[...]
    d0 = x[0:1, :] + x[1:2, :] + x[2:3, :] + 1e-6
    d1 = x[4:5, :] + x[5:6, :] + x[6:7, :] + 1e-6
    r0 = lax.rsqrt(d0)
    r1 = lax.rsqrt(d1)
    rowi = lax.broadcasted_iota(jnp.int32, x.shape, 0)
    o_ref[...] = jnp.where(rowi == 0, r0, jnp.where(rowi == 1, r1, 0.0))


def _lanes16():
    return lax.broadcasted_iota(jnp.int32, (16,), 0)


def _sc_edges_body(intsT, valsT, intsF, valsF, dpart, wv,
                   iown0, iown1, vown0, vown1, gkeep, gdup, g1i, g1v,
                   obuf, wvbuf, evjb, down0, down1, zbuf, dsh0, dsh1, gsem,
                   *, b_total, rp, nc, ns):
    cid = lax.axis_index("c")
    sid = lax.axis_index("s")
    wid = sid * nc + cid
    base = wid * rp
    nch = rp // 16
    lanes = _lanes16()
    z16 = jnp.zeros((16,), jnp.float32)

    def zb(i, c):
        zbuf[pl.ds(i * 16, 16)] = z16
        return c
    lax.fori_loop(0, b_total // 16, zb, 0)

    @pl.when(sid == 0)
    def _():
        pltpu.sync_copy(zbuf, dsh0)
        pltpu.sync_copy(zbuf, dsh1)
    plsc.subcore_barrier()

    pltpu.sync_copy(intsT.at[0, :, pl.ds(base, rp)], iown0)
    pltpu.sync_copy(intsT.at[1, :, pl.ds(base, rp)], iown1)
    pltpu.sync_copy(valsT.at[0, :, pl.ds(base, rp)], vown0)
    pltpu.sync_copy(valsT.at[1, :, pl.ds(base, rp)], vown1)

    def zd(i, c):
        down0[pl.ds(i * 16, 16)] = z16
        down1[pl.ds(i * 16, 16)] = z16
        return c
    lax.fori_loop(0, nch, zd, 0)

    for l in range(2):
        iown = iown0 if l == 0 else iown1
        vown = vown0 if l == 0 else vown1
        dsh = dsh0 if l == 0 else dsh1
        down = down0 if l == 0 else down1
        for k in range(_TOPK):
            def ob(c, carry):
                j16 = iown[k, pl.ds(c * 16, 16)]
                for kk in range(_TOPK):
                    obuf[kk, pl.ds(c * 16, 16)] = j16 + (l * 8 + kk) * b_total
                obuf[5, pl.ds(c * 16, 16)] = j16 + (l * 8 + 5) * b_total
                if l == 0:
                    for kk in range(_TOPK):
                        obuf[6 + kk, pl.ds(c * 16, 16)] = j16 + (8 + kk) * b_total
                    obuf[11, pl.ds(c * 16, 16)] = j16 + (8 + 5) * b_total
                return carry
            lax.fori_loop(0, nch, ob, 0)

            handles = []
            for kk in range(_TOPK):
                handles.append(pltpu.async_copy(intsF.at[obuf.at[kk]], gdup.at[kk], gsem))
            handles.append(pltpu.async_copy(valsF.at[obuf.at[5]], gkeep, gsem))
            if l == 0:
                for kk in range(_TOPK):
                    handles.append(pltpu.async_copy(intsF.at[obuf.at[6 + kk]], g1i.at[kk], gsem))
                    handles.append(pltpu.async_copy(valsF.at[obuf.at[6 + kk]], g1v.at[kk], gsem))
                handles.append(pltpu.async_copy(valsF.at[obuf.at[11]], g1v.at[5], gsem))
            for h in handles:
                h.wait()

            def ch(c, carry):
                sl = pl.ds(c * 16, 16)
                i16 = base + c * 16 + lanes
                j16 = iown[k, sl]
                v16 = vown[k, sl]
                valid = vown[5, sl] * gkeep[sl]
                dup_b = (gdup[0, sl] == i16)
                for kk in range(1, _TOPK):
                    dup_b = dup_b | (gdup[kk, sl] == i16)
                dupf = jnp.where(dup_b, 1.0, 0.0)
                wvbuf[k, sl] = (2.0 - dupf) * valid
                ev = v16 * valid
                down[sl] = down[sl] + ev
                evjb[sl] = ev * (1.0 - dupf)
                if l == 0:
                    fi_b = (iown1[0, sl] == j16)
                    vfi = jnp.where(iown1[0, sl] == j16, vown1[0, sl], 0.0)
                    for kk in range(1, _TOPK):
                        m = iown1[kk, sl] == j16
                        fi_b = fi_b | m
                        vfi = jnp.where(m, vown1[kk, sl], vfi)
                    fj_b = (g1i[0, sl] == i16)
                    vfj = jnp.where(g1i[0, sl] == i16, g1v[0, sl], 0.0)
                    for kk in range(1, _TOPK):
                        m = g1i[kk, sl] == i16
                        fj_b = fj_b | m
                        vfj = jnp.where(m, g1v[kk, sl], vfj)
                    in1 = jnp.where(fi_b | fj_b, 1.0, 0.0) * vown1[5, sl] * g1v[5, sl]
                    wvbuf[5 + k, sl] = in1
                    wvbuf[10 + k, sl] = jnp.where(fi_b, vfi, vfj)
                return carry
            lax.fori_loop(0, nch, ch, 0)

            pltpu.sync_copy(evjb, dsh.at[iown.at[k]], add=True)

        pltpu.sync_copy(wvbuf, wv.at[l, :, pl.ds(base, rp)])
        pltpu.sync_copy(down, dpart.at[l, 2, pl.ds(base, rp)])

    plsc.subcore_barrier()
    rp2 = b_total // ns
    sbase = sid * rp2
    pltpu.sync_copy(dsh0.at[pl.ds(sbase, rp2)], dpart.at[0, cid, pl.ds(sbase, rp2)])
    pltpu.sync_copy(dsh1.at[pl.ds(sbase, rp2)], dpart.at[1, cid, pl.ds(sbase, rp2)])


def _sc_loss_body(intsT, valsT, wv, dinv8, dinv8F, out,
                  iown0, iown1, vown0, vown1, wvb0, wvb1,
                  dvi0, dvi1, gdj, gdj1, obuf, accb, gsem,
                  *, b_total, rp, nc):
    cid = lax.axis_index("c")
    sid = lax.axis_index("s")
    wid = sid * nc + cid
    base = wid * rp
    nch = rp // 16
    z16 = jnp.zeros((16,), jnp.float32)

    pltpu.sync_copy(intsT.at[0, :, pl.ds(base, rp)], iown0)
    pltpu.sync_copy(intsT.at[1, :, pl.ds(base, rp)], iown1)
    pltpu.sync_copy(valsT.at[0, :, pl.ds(base, rp)], vown0)
    pltpu.sync_copy(valsT.at[1, :, pl.ds(base, rp)], vown1)
    pltpu.sync_copy(wv.at[0, :, pl.ds(base, rp)], wvb0)
    pltpu.sync_copy(wv.at[1, :, pl.ds(base, rp)], wvb1)
    pltpu.sync_copy(dinv8.at[0, pl.ds(base, rp)], dvi0)
    pltpu.sync_copy(dinv8.at[1, pl.ds(base, rp)], dvi1)

    s0 = z16; n0 = z16; s1 = z16; n1 = z16; cx = z16; ncx = z16
    for l in range(2):
        iown = iown0 if l == 0 else iown1
        vown = vown0 if l == 0 else vown1
        wvb = wvb0 if l == 0 else wvb1
        dvi = dvi0 if l == 0 else dvi1
        for k in range(_TOPK):
            def ob(c, carry):
                j16 = iown[k, pl.ds(c * 16, 16)]
                obuf[0, pl.ds(c * 16, 16)] = j16 + l * b_total
                if l == 0:
                    obuf[1, pl.ds(c * 16, 16)] = j16 + b_total
                return carry
            lax.fori_loop(0, nch, ob, 0)
            hs = [pltpu.async_copy(dinv8F.at[obuf.at[0]], gdj, gsem)]
            if l == 0:
                hs.append(pltpu.async_copy(dinv8F.at[obuf.at[1]], gdj1, gsem))
            for h in hs:
                h.wait()

            def ch(c, carry):
                a_s, a_n, a_c, a_nc = carry
                sl = pl.ds(c * 16, 16)
                wvk = wvb[k, sl]
                ke = dvi[sl] * vown[k, sl] * gdj[sl]
                a_s = a_s + wvk * ke * ke
                a_n = a_n + wvk
                if l == 0:
                    in1 = wvb[5 + k, sl]
                    v1e = wvb[10 + k, sl]
                    k1x = dvi1[sl] * v1e * gdj1[sl]
                    t = wvk * in1
                    a_c = a_c + t * ke * k1x
                    a_nc = a_nc + t
                return (a_s, a_n, a_c, a_nc)

            if l == 0:
                s0, n0, cx, ncx = lax.fori_loop(0, nch, ch, (s0, n0, cx, ncx))
            else:
                s1, n1, cx, ncx = lax.fori_loop(0, nch, ch, (s1, n1, cx, ncx))

    accb[pl.ds(0, 16)] = s0
    accb[pl.ds(16, 16)] = n0
    accb[pl.ds(32, 16)] = s1
    accb[pl.ds(48, 16)] = n1
    accb[pl.ds(64, 16)] = cx
    accb[pl.ds(80, 16)] = ncx
    accb[pl.ds(96, 16)] = z16
    accb[pl.ds(112, 16)] = z16
    pltpu.sync_copy(accb, out.at[wid])


def kernel(feats_final, labels, W1, W2, bn_w, bn_b, lambda_align_K, lambda_align_Z):
    L, B, D = feats_final.shape
    bm1 = 256 if B % 256 == 0 else B
    nb1 = B // bm1

    info = plsc.get_sparse_core_info()
    nc, ns = info.num_cores, info.num_subcores
    nw = nc * ns
    rp = B // nw

    xn = pl.pallas_call(
        _norm_body,
        grid=(L,),
        in_specs=[pl.BlockSpec((1, B, D), lambda l: (l, 0, 0))],
        out_specs=pl.BlockSpec((1, B, D), lambda l: (l, 0, 0)),
        out_shape=jax.ShapeDtypeStruct((L, B, D), jnp.float32),
    )(feats_final)

    labr = jnp.broadcast_to(labels[None, :], (8, B))
    labc = jnp.broadcast_to(labels[:, None], (B, 128))

    ints, vals = pl.pallas_call(
        functools.partial(_topk_body, bm=bm1, b_total=B),
        grid=(L, nb1),
        in_specs=[
            pl.BlockSpec((1, bm1, D), lambda l, b: (l, b, 0)),
            pl.BlockSpec((1, B, D), lambda l, b: (l, 0, 0)),
            pl.BlockSpec((8, B), lambda l, b: (0, 0)),
            pl.BlockSpec((bm1, 128), lambda l, b: (b, 0)),
        ],
        out_specs=[
            pl.BlockSpec((1, bm1, 8), lambda l, b: (l, b, 0)),
            pl.BlockSpec((1, bm1, 8), lambda l, b: (l, b, 0)),
        ],
        out_shape=[
            jax.ShapeDtypeStruct((L, B, 8), jnp.int32),
            jax.ShapeDtypeStruct((L, B, 8), jnp.float32),
        ],
    )(xn, xn, labr, labc)

    intsT = jnp.swapaxes(ints, 1, 2)
    valsT = jnp.swapaxes(vals, 1, 2)
    intsF = intsT.reshape(-1)
    valsF = valsT.reshape(-1)

    mesh = plsc.VectorSubcoreMesh(core_axis_name="c", subcore_axis_name="s")
    dpart, wv = pl.kernel(
        functools.partial(_sc_edges_body, b_total=B, rp=rp, nc=nc, ns=ns),
        out_type=[
            jax.ShapeDtypeStruct((L, 4, B), jnp.float32),
            jax.ShapeDtypeStruct((L, 16, B), jnp.float32),
        ],
        mesh=mesh,
        scratch_types=[
            pltpu.VMEM((8, rp), jnp.int32),
            pltpu.VMEM((8, rp), jnp.int32),
            pltpu.VMEM((8, rp), jnp.float32),
            pltpu.VMEM((8, rp), jnp.float32),
            pltpu.VMEM((rp,), jnp.float32),
            pltpu.VMEM((8, rp), jnp.int32),
            pltpu.VMEM((8, rp), jnp.int32),
            pltpu.VMEM((8, rp), jnp.float32),
            pltpu.VMEM((12, rp), jnp.int32),
            pltpu.VMEM((16, rp), jnp.float32),
            pltpu.VMEM((rp,), jnp.float32),
            pltpu.VMEM((rp,), jnp.float32),
            pltpu.VMEM((rp,), jnp.float32),
            pltpu.VMEM((B,), jnp.float32),
            pltpu.MemorySpace.VMEM_SHARED((B,), jnp.float32),
            pltpu.MemorySpace.VMEM_SHARED((B,), jnp.float32),
            pltpu.SemaphoreType.DMA,
        ],
    )(intsT, valsT, intsF, valsF)

    dinv8 = pl.pallas_call(
        _dinv_body,
        grid=(1,),
        in_specs=[pl.BlockSpec((8, B), lambda i: (0, 0))],
        out_specs=pl.BlockSpec((8, B), lambda i: (0, 0)),
        out_shape=jax.ShapeDtypeStruct((8, B), jnp.float32),
    )(dpart.reshape(8, B))

    partial = pl.kernel(
        functools.partial(_sc_loss_body, b_total=B, rp=rp, nc=nc),
        out_type=jax.ShapeDtypeStruct((nw, 128), jnp.float32),
        mesh=mesh,
        scratch_types=[
            pltpu.VMEM((8, rp), jnp.int32),
            pltpu.VMEM((8, rp), jnp.int32),
            pltpu.VMEM((8, rp), jnp.float32),
            pltpu.VMEM((8, rp), jnp.float32),
            pltpu.VMEM((16, rp), jnp.float32),
            pltpu.VMEM((16, rp), jnp.float32),
            pltpu.VMEM((rp,), jnp.float32),
            pltpu.VMEM((rp,), jnp.float32),
            pltpu.VMEM((rp,), jnp.float32),
            pltpu.VMEM((rp,), jnp.float32),
            pltpu.VMEM((2, rp), jnp.int32),
            pltpu.VMEM((128,), jnp.float32),
            pltpu.SemaphoreType.DMA,
        ],
    )(intsT, valsT, wv, dinv8, dinv8.reshape(-1))

    psum = partial.reshape(nw, 8, 16)[:, :6, :].sum(axis=(0, 2))
    num = psum[0] + psum[2] - 2.0 * psum[4]
    den = jnp.maximum(psum[1] + psum[3] - psum[5], 1e-8)
    loss_align_K = num / den
    loss_align_Z = jnp.zeros((), jnp.float32)
    loss_pga = lambda_align_K * loss_align_K + lambda_align_Z * loss_align_Z
    return (loss_align_K, loss_align_Z, loss_pga)


# speedup vs baseline: 5.7295x; 1.0150x over previous
import functools
import jax
import jax.numpy as jnp
from jax import lax
from jax.experimental import pallas as pl
from jax.experimental.pallas import tpu as pltpu, tpu_sc as plsc

_TOPK = 5
_NEG = -1e9


def _norm_body(x_ref, o_ref):
    x = x_ref[0]
    n = jnp.sqrt(jnp.sum(x * x, axis=1, keepdims=True))
    o_ref[0] = x / jnp.maximum(n, 1e-8)


def _topk_body(xb_ref, xf_ref, labr_ref, labc_ref, ints_ref, vals_ref, *, bm, b_total):
    blk = pl.program_id(1)
    xb = xb_ref[0]
    xf = xf_ref[0]
    sim = lax.dot_general(xb, xf, (((1,), (1,)), ((), ())),
                          preferred_element_type=jnp.float32)
    cols = lax.broadcasted_iota(jnp.int32, (bm, b_total), 1)
    rows_g = lax.broadcasted_iota(jnp.int32, (bm, b_total), 0) + blk * bm
    lab_row = labr_ref[0:1, :]
    lab_blk = labc_ref[:, 0:1]
    cond = (lab_blk == lab_row) & (rows_g != cols) & (sim >= 0.0)
    cand = jnp.sum((cond & (sim > 0.0)).astype(jnp.int32), axis=1)
    keepf = jnp.where(cand >= _TOPK, 1.0, 0.0)
    bits = lax.bitcast_convert_type(sim, jnp.int32)
    pb = (bits & jnp.int32(-4096)) | (b_total - 1 - cols)
    packed = jnp.where(cond, lax.bitcast_convert_type(pb, jnp.float32), _NEG)
    idx_list, val_list = [], []
    for _ in range(_TOPK):
        pmax = jnp.max(packed, axis=1)
        pbits = lax.bitcast_convert_type(pmax, jnp.int32)
        am = (b_total - 1) - (pbits & jnp.int32(4095))
        mx = lax.bitcast_convert_type(pbits & jnp.int32(-4096), jnp.float32)
        idx_list.append(am)
        val_list.append(mx)
        packed = jnp.where(packed == pmax[:, None], -2e9, packed)
    col8 = lax.broadcasted_iota(jnp.int32, (bm, 8), 1)
    ints = jnp.zeros((bm, 8), jnp.int32)
    vals = jnp.zeros((bm, 8), jnp.float32)
    for k in range(_TOPK):
        ints = jnp.where(col8 == k, idx_list[k][:, None], ints)
        vals = jnp.where(col8 == k, val_list[k][:, None], vals)
    ints = jnp.where(col8 == _TOPK, cand[:, None], ints)
    vals = jnp.where(col8 == _TOPK, keepf[:, None], vals)
    ints_ref[0] = ints
    vals_ref[0] = vals


def _dinv_body(dp_ref, o_ref):
    x = dp_ref[...]
    d0 = x[0:1, :] + x[1:2, :] + x[2:3, :] + 1e-6
    d1 = x[4:5, :] + x[5:6, :] + x[6:7, :] + 1e-6
    r0 = lax.rsqrt(d0)
    r1 = lax.rsqrt(d1)
    rowi = lax.broadcasted_iota(jnp.int32, x.shape, 0)
    o_ref[...] = jnp.where(rowi == 0, r0, jnp.where(rowi == 1, r1, 0.0))


def _lanes16():
    return lax.broadcasted_iota(jnp.int32, (16,), 0)


def _sc_edges_body(intsT, valsT, intsF, valsF, dpart, wv,
                   iown0, iown1, vown0, vown1, gkeep, gdup, g1i, g1v,
                   obuf, wvbuf, evjb, down0, down1, zbuf, dsh0, dsh1, gsem,
                   *, b_total, rp, nc, ns):
    cid = lax.axis_index("c")
    sid = lax.axis_index("s")
    wid = sid * nc + cid
    base = wid * rp
    nch = rp // 16
    lanes = _lanes16()
    z16 = jnp.zeros((16,), jnp.float32)

    def zb(i, c):
        zbuf[pl.ds(i * 16, 16)] = z16
        return c
    lax.fori_loop(0, b_total // 16, zb, 0)

    @pl.when(sid == 0)
    def _():
        pltpu.sync_copy(zbuf, dsh0)
        pltpu.sync_copy(zbuf, dsh1)
    plsc.subcore_barrier()

    pltpu.sync_copy(intsT.at[0, :, pl.ds(base, rp)], iown0)
    pltpu.sync_copy(intsT.at[1, :, pl.ds(base, rp)], iown1)
    pltpu.sync_copy(valsT.at[0, :, pl.ds(base, rp)], vown0)
    pltpu.sync_copy(valsT.at[1, :, pl.ds(base, rp)], vown1)

    def zd(i, c):
        down0[pl.ds(i * 16, 16)] = z16
        down1[pl.ds(i * 16, 16)] = z16
        return c
    lax.fori_loop(0, nch, zd, 0)

    for l in range(2):
        iown = iown0 if l == 0 else iown1
        vown = vown0 if l == 0 else vown1
        dsh = dsh0 if l == 0 else dsh1
        down = down0 if l == 0 else down1
        for k in range(_TOPK):
            def ob(c, carry):
                j16 = iown[k, pl.ds(c * 16, 16)]
                for kk in range(_TOPK):
                    obuf[kk, pl.ds(c * 16, 16)] = j16 + (l * 8 + kk) * b_total
                obuf[5, pl.ds(c * 16, 16)] = j16 + (l * 8 + 5) * b_total
                if l == 0:
                    for kk in range(_TOPK):
                        obuf[6 + kk, pl.ds(c * 16, 16)] = j16 + (8 + kk) * b_total
                    obuf[11, pl.ds(c * 16, 16)] = j16 + (8 + 5) * b_total
                return carry
            lax.fori_loop(0, nch, ob, 0)

            handles = []
            for kk in range(_TOPK):
                handles.append(pltpu.async_copy(intsF.at[obuf.at[kk]], gdup.at[kk], gsem))
            handles.append(pltpu.async_copy(valsF.at[obuf.at[5]], gkeep, gsem))
            if l == 0:
                for kk in range(_TOPK):
                    handles.append(pltpu.async_copy(intsF.at[obuf.at[6 + kk]], g1i.at[kk], gsem))
                    handles.append(pltpu.async_copy(valsF.at[obuf.at[6 + kk]], g1v.at[kk], gsem))
                handles.append(pltpu.async_copy(valsF.at[obuf.at[11]], g1v.at[5], gsem))
            for h in handles:
                h.wait()

            def ch(c, carry):
                sl = pl.ds(c * 16, 16)
                i16 = base + c * 16 + lanes
                j16 = iown[k, sl]
                v16 = vown[k, sl]
                valid = vown[5, sl] * gkeep[sl]
                dup_b = (gdup[0, sl] == i16)
                for kk in range(1, _TOPK):
                    dup_b = dup_b | (gdup[kk, sl] == i16)
                dupf = jnp.where(dup_b, 1.0, 0.0)
                wvbuf[k, sl] = (2.0 - dupf) * valid
                ev = v16 * valid
                down[sl] = down[sl] + ev
                evjb[sl] = ev * (1.0 - dupf)
                if l == 0:
                    fi_b = (iown1[0, sl] == j16)
                    vfi = jnp.where(iown1[0, sl] == j16, vown1[0, sl], 0.0)
                    for kk in range(1, _TOPK):
                        m = iown1[kk, sl] == j16
                        fi_b = fi_b | m
                        vfi = jnp.where(m, vown1[kk, sl], vfi)
                    fj_b = (g1i[0, sl] == i16)
                    vfj = jnp.where(g1i[0, sl] == i16, g1v[0, sl], 0.0)
                    for kk in range(1, _TOPK):
                        m = g1i[kk, sl] == i16
                        fj_b = fj_b | m
                        vfj = jnp.where(m, g1v[kk, sl], vfj)
                    in1 = jnp.where(fi_b | fj_b, 1.0, 0.0) * vown1[5, sl] * g1v[5, sl]
                    wvbuf[5 + k, sl] = in1
                    wvbuf[10 + k, sl] = jnp.where(fi_b, vfi, vfj)
                return carry
            lax.fori_loop(0, nch, ch, 0)

            pltpu.sync_copy(evjb, dsh.at[iown.at[k]], add=True)

        pltpu.sync_copy(wvbuf, wv.at[l, :, pl.ds(base, rp)])
        pltpu.sync_copy(down, dpart.at[l, 2, pl.ds(base, rp)])

    plsc.subcore_barrier()
    rp2 = b_total // ns
    sbase = sid * rp2
    pltpu.sync_copy(dsh0.at[pl.ds(sbase, rp2)], dpart.at[0, cid, pl.ds(sbase, rp2)])
    pltpu.sync_copy(dsh1.at[pl.ds(sbase, rp2)], dpart.at[1, cid, pl.ds(sbase, rp2)])


def _sc_loss_body(intsT, valsT, wv, dinv8, dinv8F, out,
                  iown0, iown1, vown0, vown1, wvb0, wvb1,
                  dvi0, dvi1, gdj, gdj1, obuf, accb, gsem,
                  *, b_total, rp, nc):
    cid = lax.axis_index("c")
    sid = lax.axis_index("s")
    wid = sid * nc + cid
    base = wid * rp
    nch = rp // 16
    z16 = jnp.zeros((16,), jnp.float32)

    pltpu.sync_copy(intsT.at[0, :, pl.ds(base, rp)], iown0)
    pltpu.sync_copy(intsT.at[1, :, pl.ds(base, rp)], iown1)
    pltpu.sync_copy(valsT.at[0, :, pl.ds(base, rp)], vown0)
    pltpu.sync_copy(valsT.at[1, :, pl.ds(base, rp)], vown1)
    pltpu.sync_copy(wv.at[0, :, pl.ds(base, rp)], wvb0)
    pltpu.sync_copy(wv.at[1, :, pl.ds(base, rp)], wvb1)
    pltpu.sync_copy(dinv8.at[0, pl.ds(base, rp)], dvi0)
    pltpu.sync_copy(dinv8.at[1, pl.ds(base, rp)], dvi1)

    s0 = z16; n0 = z16; s1 = z16; n1 = z16; cx = z16; ncx = z16
    for l in range(2):
        iown = iown0 if l == 0 else iown1
        vown = vown0 if l == 0 else vown1
        wvb = wvb0 if l == 0 else wvb1
        dvi = dvi0 if l == 0 else dvi1
        for k in range(_TOPK):
            def ob(c, carry):
                j16 = iown[k, pl.ds(c * 16, 16)]
                obuf[0, pl.ds(c * 16, 16)] = j16 + l * b_total
                if l == 0:
                    obuf[1, pl.ds(c * 16, 16)] = j16 + b_total
                return carry
            lax.fori_loop(0, nch, ob, 0)
            hs = [pltpu.async_copy(dinv8F.at[obuf.at[0]], gdj, gsem)]
            if l == 0:
                hs.append(pltpu.async_copy(dinv8F.at[obuf.at[1]], gdj1, gsem))
            for h in hs:
                h.wait()

            def ch(c, carry):
                a_s, a_n, a_c, a_nc = carry
                sl = pl.ds(c * 16, 16)
                wvk = wvb[k, sl]
                ke = dvi[sl] * vown[k, sl] * gdj[sl]
                a_s = a_s + wvk * ke * ke
                a_n = a_n + wvk
                if l == 0:
                    in1 = wvb[5 + k, sl]
                    v1e = wvb[10 + k, sl]
                    k1x = dvi1[sl] * v1e * gdj1[sl]
                    t = wvk * in1
                    a_c = a_c + t * ke * k1x
                    a_nc = a_nc + t
                return (a_s, a_n, a_c, a_nc)

            if l == 0:
                s0, n0, cx, ncx = lax.fori_loop(0, nch, ch, (s0, n0, cx, ncx))
            else:
                s1, n1, cx, ncx = lax.fori_loop(0, nch, ch, (s1, n1, cx, ncx))

    accb[pl.ds(0, 16)] = s0
    accb[pl.ds(16, 16)] = n0
    accb[pl.ds(32, 16)] = s1
    accb[pl.ds(48, 16)] = n1
    accb[pl.ds(64, 16)] = cx
    accb[pl.ds(80, 16)] = ncx
    accb[pl.ds(96, 16)] = z16
    accb[pl.ds(112, 16)] = z16
    pltpu.sync_copy(accb, out.at[wid])


def kernel(feats_final, labels, W1, W2, bn_w, bn_b, lambda_align_K, lambda_align_Z):
    L, B, D = feats_final.shape
    bm1 = 512 if B % 512 == 0 else B
    nb1 = B // bm1

    info = plsc.get_sparse_core_info()
    nc, ns = info.num_cores, info.num_subcores
    nw = nc * ns
    rp = B // nw

    xn = pl.pallas_call(
        _norm_body,
        grid=(L,),
        in_specs=[pl.BlockSpec((1, B, D), lambda l: (l, 0, 0))],
        out_specs=pl.BlockSpec((1, B, D), lambda l: (l, 0, 0)),
        out_shape=jax.ShapeDtypeStruct((L, B, D), jnp.float32),
    )(feats_final)

    labr = jnp.broadcast_to(labels[None, :], (8, B))
    labc = jnp.broadcast_to(labels[:, None], (B, 128))

    ints, vals = pl.pallas_call(
        functools.partial(_topk_body, bm=bm1, b_total=B),
        grid=(L, nb1),
        in_specs=[
            pl.BlockSpec((1, bm1, D), lambda l, b: (l, b, 0)),
            pl.BlockSpec((1, B, D), lambda l, b: (l, 0, 0)),
            pl.BlockSpec((8, B), lambda l, b: (0, 0)),
            pl.BlockSpec((bm1, 128), lambda l, b: (b, 0)),
        ],
        out_specs=[
            pl.BlockSpec((1, bm1, 8), lambda l, b: (l, b, 0)),
            pl.BlockSpec((1, bm1, 8), lambda l, b: (l, b, 0)),
        ],
        out_shape=[
            jax.ShapeDtypeStruct((L, B, 8), jnp.int32),
            jax.ShapeDtypeStruct((L, B, 8), jnp.float32),
        ],
    )(xn, xn, labr, labc)

    intsT = jnp.swapaxes(ints, 1, 2)
    valsT = jnp.swapaxes(vals, 1, 2)
    intsF = intsT.reshape(-1)
    valsF = valsT.reshape(-1)

    mesh = plsc.VectorSubcoreMesh(core_axis_name="c", subcore_axis_name="s")
    dpart, wv = pl.kernel(
        functools.partial(_sc_edges_body, b_total=B, rp=rp, nc=nc, ns=ns),
        out_type=[
            jax.ShapeDtypeStruct((L, 4, B), jnp.float32),
            jax.ShapeDtypeStruct((L, 16, B), jnp.float32),
        ],
        mesh=mesh,
        scratch_types=[
            pltpu.VMEM((8, rp), jnp.int32),
            pltpu.VMEM((8, rp), jnp.int32),
            pltpu.VMEM((8, rp), jnp.float32),
            pltpu.VMEM((8, rp), jnp.float32),
            pltpu.VMEM((rp,), jnp.float32),
            pltpu.VMEM((8, rp), jnp.int32),
            pltpu.VMEM((8, rp), jnp.int32),
            pltpu.VMEM((8, rp), jnp.float32),
            pltpu.VMEM((12, rp), jnp.int32),
            pltpu.VMEM((16, rp), jnp.float32),
            pltpu.VMEM((rp,), jnp.float32),
            pltpu.VMEM((rp,), jnp.float32),
            pltpu.VMEM((rp,), jnp.float32),
            pltpu.VMEM((B,), jnp.float32),
            pltpu.MemorySpace.VMEM_SHARED((B,), jnp.float32),
            pltpu.MemorySpace.VMEM_SHARED((B,), jnp.float32),
            pltpu.SemaphoreType.DMA,
        ],
    )(intsT, valsT, intsF, valsF)

    dinv8 = pl.pallas_call(
        _dinv_body,
        grid=(1,),
        in_specs=[pl.BlockSpec((8, B), lambda i: (0, 0))],
        out_specs=pl.BlockSpec((8, B), lambda i: (0, 0)),
        out_shape=jax.ShapeDtypeStruct((8, B), jnp.float32),
    )(dpart.reshape(8, B))

    partial = pl.kernel(
        functools.partial(_sc_loss_body, b_total=B, rp=rp, nc=nc),
        out_type=jax.ShapeDtypeStruct((nw, 128), jnp.float32),
        mesh=mesh,
        scratch_types=[
            pltpu.VMEM((8, rp), jnp.int32),
            pltpu.VMEM((8, rp), jnp.int32),
            pltpu.VMEM((8, rp), jnp.float32),
            pltpu.VMEM((8, rp), jnp.float32),
            pltpu.VMEM((16, rp), jnp.float32),
            pltpu.VMEM((16, rp), jnp.float32),
            pltpu.VMEM((rp,), jnp.float32),
            pltpu.VMEM((rp,), jnp.float32),
            pltpu.VMEM((rp,), jnp.float32),
            pltpu.VMEM((rp,), jnp.float32),
            pltpu.VMEM((2, rp), jnp.int32),
            pltpu.VMEM((128,), jnp.float32),
            pltpu.SemaphoreType.DMA,
        ],
    )(intsT, valsT, wv, dinv8, dinv8.reshape(-1))

    psum = partial.reshape(nw, 8, 16)[:, :6, :].sum(axis=(0, 2))
    num = psum[0] + psum[2] - 2.0 * psum[4]
    den = jnp.maximum(psum[1] + psum[3] - psum[5], 1e-8)
    loss_align_K = num / den
    loss_align_Z = jnp.zeros((), jnp.float32)
    loss_pga = lambda_align_K * loss_align_K + lambda_align_Z * loss_align_Z
    return (loss_align_K, loss_align_Z, loss_pga)


# speedup vs baseline: 6.4474x; 1.1253x over previous
import functools
import jax
import jax.numpy as jnp
from jax import lax
from jax.experimental import pallas as pl
from jax.experimental.pallas import tpu as pltpu, tpu_sc as plsc

_TOPK = 5
_NEG = -1e9


def _norm_body(x_ref, o_ref):
    x = x_ref[0]
    n = jnp.sqrt(jnp.sum(x * x, axis=1, keepdims=True))
    o_ref[0] = x / jnp.maximum(n, 1e-8)


def _topk_body(xb_ref, xf_ref, labr_ref, labc_ref, ints_ref, vals_ref, *, bm, b_total):
    blk = pl.program_id(1)
    xb = xb_ref[0]
    xf = xf_ref[0]
    sim = lax.dot_general(xb, xf, (((1,), (1,)), ((), ())),
                          preferred_element_type=jnp.float32)
    cols = lax.broadcasted_iota(jnp.int32, (bm, b_total), 1)
    rows_g = lax.broadcasted_iota(jnp.int32, (bm, b_total), 0) + blk * bm
    lab_row = labr_ref[0:1, :]
    lab_blk = labc_ref[:, 0:1]
    cond = (lab_blk == lab_row) & (rows_g != cols) & (sim >= 0.0)
    cand = jnp.sum((cond & (sim > 0.0)).astype(jnp.int32), axis=1)
    keepf = jnp.where(cand >= _TOPK, 1.0, 0.0)
    bits = lax.bitcast_convert_type(sim, jnp.int32)
    pb = (bits & jnp.int32(-4096)) | (b_total - 1 - cols)
    packed = jnp.where(cond, lax.bitcast_convert_type(pb, jnp.float32), _NEG)
    idx_list, val_list = [], []
    for _ in range(_TOPK):
        pmax = jnp.max(packed, axis=1)
        pbits = lax.bitcast_convert_type(pmax, jnp.int32)
        am = (b_total - 1) - (pbits & jnp.int32(4095))
        mx = lax.bitcast_convert_type(pbits & jnp.int32(-4096), jnp.float32)
        idx_list.append(am)
        val_list.append(mx)
        packed = jnp.where(packed == pmax[:, None], -2e9, packed)
    col8 = lax.broadcasted_iota(jnp.int32, (bm, 8), 1)
    ints = jnp.zeros((bm, 8), jnp.int32)
    vals = jnp.zeros((bm, 8), jnp.float32)
    for k in range(_TOPK):
        ints = jnp.where(col8 == k, idx_list[k][:, None], ints)
        vals = jnp.where(col8 == k, val_list[k][:, None], vals)
    ints = jnp.where(col8 == _TOPK, cand[:, None], ints)
    vals = jnp.where(col8 == _TOPK, keepf[:, None], vals)
    ints_ref[0] = ints
    vals_ref[0] = vals


def _lanes16():
    return lax.broadcasted_iota(jnp.int32, (16,), 0)


def _sc_edges_body(intsT, valsT, intsF, valsF, dpart, wv,
                   iown0, iown1, vown0, vown1, gkeep, gdup, g1i, g1v,
                   obuf, wvbuf, evjb, down0, down1, zbuf, dsh0, dsh1, gsem,
                   sf_i, sf_v,
                   *, b_total, rp, nc, ns):
    cid = lax.axis_index("c")
    sid = lax.axis_index("s")
    wid = sid * nc + cid
    base = wid * rp
    nch = rp // 16
    lanes = _lanes16()
    z16 = jnp.zeros((16,), jnp.float32)

    def zb(i, c):
        zbuf[pl.ds(i * 16, 16)] = z16
        return c
    lax.fori_loop(0, b_total // 16, zb, 0)

    tw = (16 * b_total) // ns
    pltpu.sync_copy(intsF.at[pl.ds(sid * tw, tw)], sf_i.at[pl.ds(sid * tw, tw)])
    pltpu.sync_copy(valsF.at[pl.ds(sid * tw, tw)], sf_v.at[pl.ds(sid * tw, tw)])
    @pl.when(sid == 0)
    def _():
        pltpu.sync_copy(zbuf, dsh0)
        pltpu.sync_copy(zbuf, dsh1)
    plsc.subcore_barrier()

    pltpu.sync_copy(intsT.at[0, :, pl.ds(base, rp)], iown0)
    pltpu.sync_copy(intsT.at[1, :, pl.ds(base, rp)], iown1)
    pltpu.sync_copy(valsT.at[0, :, pl.ds(base, rp)], vown0)
    pltpu.sync_copy(valsT.at[1, :, pl.ds(base, rp)], vown1)

    def zd(i, c):
        down0[pl.ds(i * 16, 16)] = z16
        down1[pl.ds(i * 16, 16)] = z16
        return c
    lax.fori_loop(0, nch, zd, 0)

    for l in range(2):
        iown = iown0 if l == 0 else iown1
        vown = vown0 if l == 0 else vown1
        dsh = dsh0 if l == 0 else dsh1
        down = down0 if l == 0 else down1
        for k in range(_TOPK):
            def ob(c, carry):
                j16 = iown[k, pl.ds(c * 16, 16)]
                for kk in range(_TOPK):
                    obuf[kk, pl.ds(c * 16, 16)] = j16 + (l * 8 + kk) * b_total
                obuf[5, pl.ds(c * 16, 16)] = j16 + (l * 8 + 5) * b_total
                if l == 0:
                    for kk in range(_TOPK):
                        obuf[6 + kk, pl.ds(c * 16, 16)] = j16 + (8 + kk) * b_total
                    obuf[11, pl.ds(c * 16, 16)] = j16 + (8 + 5) * b_total
                return carry
            lax.fori_loop(0, nch, ob, 0)

            handles = []
            for kk in range(_TOPK):
                handles.append(pltpu.async_copy(sf_i.at[obuf.at[kk]], gdup.at[kk], gsem))
            handles.append(pltpu.async_copy(sf_v.at[obuf.at[5]], gkeep, gsem))
            if l == 0:
                for kk in range(_TOPK):
                    handles.append(pltpu.async_copy(sf_i.at[obuf.at[6 + kk]], g1i.at[kk], gsem))
                    handles.append(pltpu.async_copy(sf_v.at[obuf.at[6 + kk]], g1v.at[kk], gsem))
                handles.append(pltpu.async_copy(sf_v.at[obuf.at[11]], g1v.at[5], gsem))
            for h in handles:
                h.wait()

            def ch(c, carry):
                sl = pl.ds(c * 16, 16)
                i16 = base + c * 16 + lanes
                j16 = iown[k, sl]
                v16 = vown[k, sl]
                valid = vown[5, sl] * gkeep[sl]
                dup_b = (gdup[0, sl] == i16)
                for kk in range(1, _TOPK):
                    dup_b = dup_b | (gdup[kk, sl] == i16)
                dupf = jnp.where(dup_b, 1.0, 0.0)
                wvbuf[k, sl] = (2.0 - dupf) * valid
                ev = v16 * valid
                down[sl] = down[sl] + ev
                evjb[sl] = ev * (1.0 - dupf)
                if l == 0:
                    fi_b = (iown1[0, sl] == j16)
                    vfi = jnp.where(iown1[0, sl] == j16, vown1[0, sl], 0.0)
                    for kk in range(1, _TOPK):
                        m = iown1[kk, sl] == j16
                        fi_b = fi_b | m
                        vfi = jnp.where(m, vown1[kk, sl], vfi)
                    fj_b = (g1i[0, sl] == i16)
                    vfj = jnp.where(g1i[0, sl] == i16, g1v[0, sl], 0.0)
                    for kk in range(1, _TOPK):
                        m = g1i[kk, sl] == i16
                        fj_b = fj_b | m
                        vfj = jnp.where(m, g1v[kk, sl], vfj)
                    in1 = jnp.where(fi_b | fj_b, 1.0, 0.0) * vown1[5, sl] * g1v[5, sl]
                    wvbuf[5 + k, sl] = in1
                    wvbuf[10 + k, sl] = jnp.where(fi_b, vfi, vfj)
                return carry
            lax.fori_loop(0, nch, ch, 0)

            pltpu.sync_copy(evjb, dsh.at[iown.at[k]], add=True)

        pltpu.sync_copy(wvbuf, wv.at[l, :, pl.ds(base, rp)])
        pltpu.sync_copy(down, dpart.at[l, 2, pl.ds(base, rp)])

    plsc.subcore_barrier()
    rp2 = b_total // ns
    sbase = sid * rp2
    pltpu.sync_copy(dsh0.at[pl.ds(sbase, rp2)], dpart.at[0, cid, pl.ds(sbase, rp2)])
    pltpu.sync_copy(dsh1.at[pl.ds(sbase, rp2)], dpart.at[1, cid, pl.ds(sbase, rp2)])


def _sc_loss_body(intsT, valsT, wv, dpart, out,
                  iown0, iown1, vown0, vown1, wvb0, wvb1,
                  dvi0, dvi1, gdj, gdj1, obuf, accb, gsem,
                  dbuf, dtmp, sdinv0, sdinv1,
                  *, b_total, rp, nc, ns):
    cid = lax.axis_index("c")
    sid = lax.axis_index("s")
    wid = sid * nc + cid
    base = wid * rp
    nch = rp // 16
    z16 = jnp.zeros((16,), jnp.float32)

    rp2 = b_total // ns
    sb = sid * rp2
    for l in range(2):
        sdinv = sdinv0 if l == 0 else sdinv1
        pltpu.sync_copy(dpart.at[l, 0, pl.ds(sb, rp2)], dbuf)
        pltpu.sync_copy(dpart.at[l, 1, pl.ds(sb, rp2)], dtmp)

        def ac(i, c):
            sl = pl.ds(i * 16, 16)
            dbuf[sl] = dbuf[sl] + dtmp[sl]
            return c
        lax.fori_loop(0, rp2 // 16, ac, 0)
        pltpu.sync_copy(dpart.at[l, 2, pl.ds(sb, rp2)], dtmp)

        def rs(i, c):
            sl = pl.ds(i * 16, 16)
            d = dbuf[sl] + dtmp[sl] + 1e-6
            ii = lax.bitcast_convert_type(d, jnp.int32)
            y = lax.bitcast_convert_type(
                jnp.int32(0x5F3759DF) - lax.shift_right_logical(ii, 1), jnp.float32)
            for _ in range(4):
                y = y * (1.5 - 0.5 * d * y * y)
            dbuf[sl] = y
            return c
        lax.fori_loop(0, rp2 // 16, rs, 0)
        pltpu.sync_copy(dbuf, sdinv.at[pl.ds(sb, rp2)])

    pltpu.sync_copy(intsT.at[0, :, pl.ds(base, rp)], iown0)
    pltpu.sync_copy(intsT.at[1, :, pl.ds(base, rp)], iown1)
    pltpu.sync_copy(valsT.at[0, :, pl.ds(base, rp)], vown0)
    pltpu.sync_copy(valsT.at[1, :, pl.ds(base, rp)], vown1)
    pltpu.sync_copy(wv.at[0, :, pl.ds(base, rp)], wvb0)
    pltpu.sync_copy(wv.at[1, :, pl.ds(base, rp)], wvb1)
    plsc.subcore_barrier()
    pltpu.sync_copy(sdinv0.at[pl.ds(base, rp)], dvi0)
    pltpu.sync_copy(sdinv1.at[pl.ds(base, rp)], dvi1)

    s0 = z16; n0 = z16; s1 = z16; n1 = z16; cx = z16; ncx = z16
    for l in range(2):
        iown = iown0 if l == 0 else iown1
        vown = vown0 if l == 0 else vown1
        wvb = wvb0 if l == 0 else wvb1
        dvi = dvi0 if l == 0 else dvi1
        for k in range(_TOPK):
            hs = [pltpu.async_copy((sdinv0 if l == 0 else sdinv1).at[iown.at[k]],
                                   gdj, gsem)]
            if l == 0:
                hs.append(pltpu.async_copy(sdinv1.at[iown.at[k]], gdj1, gsem))
            for h in hs:
                h.wait()

            def ch(c, carry):
                a_s, a_n, a_c, a_nc = carry
                sl = pl.ds(c * 16, 16)
                wvk = wvb[k, sl]
                ke = dvi[sl] * vown[k, sl] * gdj[sl]
                a_s = a_s + wvk * ke * ke
                a_n = a_n + wvk
                if l == 0:
                    in1 = wvb[5 + k, sl]
                    v1e = wvb[10 + k, sl]
                    k1x = dvi1[sl] * v1e * gdj1[sl]
                    t = wvk * in1
                    a_c = a_c + t * ke * k1x
                    a_nc = a_nc + t
                return (a_s, a_n, a_c, a_nc)

            if l == 0:
                s0, n0, cx, ncx = lax.fori_loop(0, nch, ch, (s0, n0, cx, ncx))
            else:
                s1, n1, cx, ncx = lax.fori_loop(0, nch, ch, (s1, n1, cx, ncx))

    accb[pl.ds(0, 16)] = s0
    accb[pl.ds(16, 16)] = n0
    accb[pl.ds(32, 16)] = s1
    accb[pl.ds(48, 16)] = n1
    accb[pl.ds(64, 16)] = cx
    accb[pl.ds(80, 16)] = ncx
    accb[pl.ds(96, 16)] = z16
    accb[pl.ds(112, 16)] = z16
    pltpu.sync_copy(accb, out.at[wid])


def kernel(feats_final, labels, W1, W2, bn_w, bn_b, lambda_align_K, lambda_align_Z):
    L, B, D = feats_final.shape
    bm1 = 512 if B % 512 == 0 else B
    nb1 = B // bm1

    info = plsc.get_sparse_core_info()
    nc, ns = info.num_cores, info.num_subcores
    nw = nc * ns
    rp = B // nw

    xn = pl.pallas_call(
        _norm_body,
        grid=(L,),
        in_specs=[pl.BlockSpec((1, B, D), lambda l: (l, 0, 0))],
        out_specs=pl.BlockSpec((1, B, D), lambda l: (l, 0, 0)),
        out_shape=jax.ShapeDtypeStruct((L, B, D), jnp.float32),
    )(feats_final)

    labr = jnp.broadcast_to(labels[None, :], (8, B))
    labc = jnp.broadcast_to(labels[:, None], (B, 128))

    ints, vals = pl.pallas_call(
        functools.partial(_topk_body, bm=bm1, b_total=B),
        grid=(L, nb1),
        in_specs=[
            pl.BlockSpec((1, bm1, D), lambda l, b: (l, b, 0)),
            pl.BlockSpec((1, B, D), lambda l, b: (l, 0, 0)),
            pl.BlockSpec((8, B), lambda l, b: (0, 0)),
            pl.BlockSpec((bm1, 128), lambda l, b: (b, 0)),
        ],
        out_specs=[
            pl.BlockSpec((1, bm1, 8), lambda l, b: (l, b, 0)),
            pl.BlockSpec((1, bm1, 8), lambda l, b: (l, b, 0)),
        ],
        out_shape=[
            jax.ShapeDtypeStruct((L, B, 8), jnp.int32),
            jax.ShapeDtypeStruct((L, B, 8), jnp.float32),
        ],
    )(xn, xn, labr, labc)

    intsT = jnp.swapaxes(ints, 1, 2)
    valsT = jnp.swapaxes(vals, 1, 2)
    intsF = intsT.reshape(-1)
    valsF = valsT.reshape(-1)

    mesh = plsc.VectorSubcoreMesh(core_axis_name="c", subcore_axis_name="s")
    dpart, wv = pl.kernel(
        functools.partial(_sc_edges_body, b_total=B, rp=rp, nc=nc, ns=ns),
        out_type=[
            jax.ShapeDtypeStruct((L, 4, B), jnp.float32),
            jax.ShapeDtypeStruct((L, 16, B), jnp.float32),
        ],
        mesh=mesh,
        scratch_types=[
            pltpu.VMEM((8, rp), jnp.int32),
            pltpu.VMEM((8, rp), jnp.int32),
            pltpu.VMEM((8, rp), jnp.float32),
            pltpu.VMEM((8, rp), jnp.float32),
            pltpu.VMEM((rp,), jnp.float32),
            pltpu.VMEM((8, rp), jnp.int32),
            pltpu.VMEM((8, rp), jnp.int32),
            pltpu.VMEM((8, rp), jnp.float32),
            pltpu.VMEM((12, rp), jnp.int32),
            pltpu.VMEM((16, rp), jnp.float32),
            pltpu.VMEM((rp,), jnp.float32),
            pltpu.VMEM((rp,), jnp.float32),
            pltpu.VMEM((rp,), jnp.float32),
            pltpu.VMEM((B,), jnp.float32),
            pltpu.MemorySpace.VMEM_SHARED((B,), jnp.float32),
            pltpu.MemorySpace.VMEM_SHARED((B,), jnp.float32),
            pltpu.SemaphoreType.DMA,
            pltpu.MemorySpace.VMEM_SHARED((16 * B,), jnp.int32),
            pltpu.MemorySpace.VMEM_SHARED((16 * B,), jnp.float32),
        ],
    )(intsT, valsT, intsF, valsF)

    partial = pl.kernel(
        functools.partial(_sc_loss_body, b_total=B, rp=rp, nc=nc, ns=ns),
        out_type=jax.ShapeDtypeStruct((nw, 128), jnp.float32),
        mesh=mesh,
        scratch_types=[
            pltpu.VMEM((8, rp), jnp.int32),
            pltpu.VMEM((8, rp), jnp.int32),
            pltpu.VMEM((8, rp), jnp.float32),
            pltpu.VMEM((8, rp), jnp.float32),
            pltpu.VMEM((16, rp), jnp.float32),
            pltpu.VMEM((16, rp), jnp.float32),
            pltpu.VMEM((rp,), jnp.float32),
            pltpu.VMEM((rp,), jnp.float32),
            pltpu.VMEM((rp,), jnp.float32),
            pltpu.VMEM((rp,), jnp.float32),
            pltpu.VMEM((2, rp), jnp.int32),
            pltpu.VMEM((128,), jnp.float32),
            pltpu.SemaphoreType.DMA,
            pltpu.VMEM((B // ns,), jnp.float32),
            pltpu.VMEM((B // ns,), jnp.float32),
            pltpu.MemorySpace.VMEM_SHARED((B,), jnp.float32),
            pltpu.MemorySpace.VMEM_SHARED((B,), jnp.float32),
        ],
    )(intsT, valsT, wv, dpart)

    psum = partial.reshape(nw, 8, 16)[:, :6, :].sum(axis=(0, 2))
    num = psum[0] + psum[2] - 2.0 * psum[4]
    den = jnp.maximum(psum[1] + psum[3] - psum[5], 1e-8)
    loss_align_K = num / den
    loss_align_Z = jnp.zeros((), jnp.float32)
    loss_pga = lambda_align_K * loss_align_K + lambda_align_Z * loss_align_Z
    return (loss_align_K, loss_align_Z, loss_pga)
